# Initial kernel scaffold; baseline (speedup 1.0000x reference)
import jax
import jax.numpy as jnp
from jax.experimental import pallas as pl


def kernel(x_temporal, x_topology, Wc, bc, Wx1, Wh1, b1, Wx2, Wh2, b2, W_in, Wself, Wnbr, bsp, conv_w, conv_b, Wcls, bcls):
    raise NotImplementedError("write your pallas kernel here")


# speedup vs baseline: 223.5235x; 223.5235x over previous
import jax
import jax.numpy as jnp
from jax.experimental import pallas as pl
from jax.experimental.pallas import tpu as pltpu

B, T, F = 8, 256, 65
N, K = 2048, 16
H, HID, OUT = 128, 64, 8
L = 4


def _gru_pipeline_kernel(x_ref, wx1_ref, wh1_ref, b1_ref, wx2_ref, wh2_ref,
                         b2_ref, win_ref, wself_ref, wnbr_ref, bsp_ref,
                         cwt_ref, convb_ref, wcls_ref, bcls_ref,
                         out_ref, gx1_ref):
    gx1 = jnp.dot(x_ref[:], wx1_ref[:], preferred_element_type=jnp.float32)
    gx1_ref[:] = (gx1 + b1_ref[:]).reshape(T, B, 3 * H)

    wh1 = wh1_ref[:]
    wx2 = wx2_ref[:]
    wh2 = wh2_ref[:]
    b2 = b2_ref[:]

    def gru_gates(g, gh, h):
        z = jax.nn.sigmoid(g[:, 0:H] + gh[:, 0:H])
        r = jax.nn.sigmoid(g[:, H:2 * H] + gh[:, H:2 * H])
        n = jnp.tanh(g[:, 2 * H:3 * H] + r * gh[:, 2 * H:3 * H])
        return (1.0 - z) * n + z * h

    def step(t, carry):
        h1, h2 = carry
        g1 = gx1_ref[t]
        gh1 = jnp.dot(h1, wh1, preferred_element_type=jnp.float32)
        h1n = gru_gates(g1, gh1, h1)
        g2 = jnp.dot(h1n, wx2, preferred_element_type=jnp.float32) + b2
        gh2 = jnp.dot(h2, wh2, preferred_element_type=jnp.float32)
        h2n = gru_gates(g2, gh2, h2)
        return (h1n, h2n)

    h0 = jnp.zeros((B, H), jnp.float32)
    _, hT = jax.lax.fori_loop(0, T, step, (h0, h0))

    hv = jnp.dot(hT, win_ref[:], preferred_element_type=jnp.float32)
    for l in range(L):
        tm = jnp.dot(hv, wnbr_ref[l], preferred_element_type=jnp.float32)
        sf = jnp.dot(hv, wself_ref[l], preferred_element_type=jnp.float32)
        hv = jax.nn.relu(sf + float(K) * tm + bsp_ref[l:l + 1, :])
    s = jnp.mean(hv, axis=1, keepdims=True)

    w0 = cwt_ref[0]
    w1 = cwt_ref[1]
    w2 = cwt_ref[2]
    cb = convb_ref[:]
    s_all = jnp.dot(hT, w0 + w1 + w2, preferred_element_type=jnp.float32) + cb
    s_lft = jnp.dot(hT, w1 + w2, preferred_element_type=jnp.float32) + cb
    s_rgt = jnp.dot(hT, w0 + w1, preferred_element_type=jnp.float32) + cb
    tc = (float(N - 2) * jax.nn.relu(s_all) + jax.nn.relu(s_lft)
          + jax.nn.relu(s_rgt)) * (1.0 / float(N))

    w_tc = wcls_ref[0:HID, :]
    w_sp = jnp.sum(wcls_ref[HID:, :], axis=0, keepdims=True)
    out_ref[:] = (jnp.dot(tc, w_tc, preferred_element_type=jnp.float32)
                  + s * w_sp + bcls_ref[:])


def kernel(x_temporal, x_topology, Wc, bc, Wx1, Wh1, b1, Wx2, Wh2, b2,
           W_in, Wself, Wnbr, bsp, conv_w, conv_b, Wcls, bcls):
    x2d = jnp.swapaxes(x_temporal, 0, 1).reshape(T * B, F)
    cwt = jnp.transpose(conv_w, (2, 1, 0))
    return pl.pallas_call(
        _gru_pipeline_kernel,
        out_shape=jax.ShapeDtypeStruct((B, OUT), jnp.float32),
        scratch_shapes=[pltpu.VMEM((T, B, 3 * H), jnp.float32)],
    )(x2d, Wx1, Wh1, b1.reshape(1, -1), Wx2, Wh2, b2.reshape(1, -1),
      W_in, Wself, Wnbr, bsp, cwt, conv_b.reshape(1, -1), Wcls,
      bcls.reshape(1, -1))


# speedup vs baseline: 335.4952x; 1.5009x over previous
import jax
import jax.numpy as jnp
from jax.experimental import pallas as pl
from jax.experimental.pallas import tpu as pltpu

B, T, F = 8, 256, 65
N, K = 2048, 16
H, HID, OUT = 128, 64, 8
L = 4


def _gru_pipeline_kernel(x_ref, wx1_ref, wh1_ref, b1_ref, wx2_ref, wh2_ref,
                         b2_ref, win_ref, wself_ref, wnbr_ref, bsp_ref,
                         cwt_ref, convb_ref, wcls_ref, bcls_ref,
                         out_ref, gx1_ref):
    gx1 = jnp.dot(x_ref[:], wx1_ref[:], preferred_element_type=jnp.float32)
    gx1_ref[:] = (gx1 + b1_ref[:]).reshape(T, B, 3 * H)

    wh1 = wh1_ref[:]
    wx2 = wx2_ref[:]
    wh2 = wh2_ref[:]
    b2 = b2_ref[:]

    def gru_gates(g, gh, h):
        z = jax.nn.sigmoid(g[:, 0:H] + gh[:, 0:H])
        r = jax.nn.sigmoid(g[:, H:2 * H] + gh[:, H:2 * H])
        n = jnp.tanh(g[:, 2 * H:3 * H] + r * gh[:, 2 * H:3 * H])
        return (1.0 - z) * n + z * h

    def l1_step(t, h1):
        g1 = gx1_ref[t]
        gh1 = jnp.dot(h1, wh1, preferred_element_type=jnp.float32)
        return gru_gates(g1, gh1, h1)

    def l2_step(h1, h2):
        g2 = jnp.dot(h1, wx2, preferred_element_type=jnp.float32) + b2
        gh2 = jnp.dot(h2, wh2, preferred_element_type=jnp.float32)
        return gru_gates(g2, gh2, h2)

    def step(t, carry):
        h1, h2 = carry
        return (l1_step(t, h1), l2_step(h1, h2))

    h0 = jnp.zeros((B, H), jnp.float32)
    h1 = l1_step(0, h0)
    h1, h2 = jax.lax.fori_loop(1, T, step, (h1, h0))
    hT = l2_step(h1, h2)

    hv = jnp.dot(hT, win_ref[:], preferred_element_type=jnp.float32)
    for l in range(L):
        tm = jnp.dot(hv, wnbr_ref[l], preferred_element_type=jnp.float32)
        sf = jnp.dot(hv, wself_ref[l], preferred_element_type=jnp.float32)
        hv = jax.nn.relu(sf + float(K) * tm + bsp_ref[l:l + 1, :])
    s = jnp.mean(hv, axis=1, keepdims=True)

    w0 = cwt_ref[0]
    w1 = cwt_ref[1]
    w2 = cwt_ref[2]
    cb = convb_ref[:]
    s_all = jnp.dot(hT, w0 + w1 + w2, preferred_element_type=jnp.float32) + cb
    s_lft = jnp.dot(hT, w1 + w2, preferred_element_type=jnp.float32) + cb
    s_rgt = jnp.dot(hT, w0 + w1, preferred_element_type=jnp.float32) + cb
    tc = (float(N - 2) * jax.nn.relu(s_all) + jax.nn.relu(s_lft)
          + jax.nn.relu(s_rgt)) * (1.0 / float(N))

    w_tc = wcls_ref[0:HID, :]
    w_sp = jnp.sum(wcls_ref[HID:, :], axis=0, keepdims=True)
    out_ref[:] = (jnp.dot(tc, w_tc, preferred_element_type=jnp.float32)
                  + s * w_sp + bcls_ref[:])


def kernel(x_temporal, x_topology, Wc, bc, Wx1, Wh1, b1, Wx2, Wh2, b2,
           W_in, Wself, Wnbr, bsp, conv_w, conv_b, Wcls, bcls):
    x2d = jnp.swapaxes(x_temporal, 0, 1).reshape(T * B, F)
    cwt = jnp.transpose(conv_w, (2, 1, 0))
    return pl.pallas_call(
        _gru_pipeline_kernel,
        out_shape=jax.ShapeDtypeStruct((B, OUT), jnp.float32),
        scratch_shapes=[pltpu.VMEM((T, B, 3 * H), jnp.float32)],
    )(x2d, Wx1, Wh1, b1.reshape(1, -1), Wx2, Wh2, b2.reshape(1, -1),
      W_in, Wself, Wnbr, bsp, cwt, conv_b.reshape(1, -1), Wcls,
      bcls.reshape(1, -1))
